# Initial kernel scaffold; baseline (speedup 1.0000x reference)
import jax
import jax.numpy as jnp
from jax.experimental import pallas as pl


def kernel(x):
    raise NotImplementedError("write your pallas kernel here")


# speedup vs baseline: 1.4339x; 1.4339x over previous
import dataclasses

import jax
import jax.numpy as jnp
import numpy as np
from jax import lax
from jax.experimental import pallas as pl
from jax.experimental.pallas import tpu as pltpu
from jax.experimental.pallas import tpu_sc as plsc

ROWS = 128
N = 32768
L = 16
NUM_WORKERS = 32
ROWS_PER_WORKER = ROWS // NUM_WORKERS

_MIN_I32 = np.int32(-(2**31))


def _sort_body(x_hbm, vals_hbm, idx_hbm, keys, pa, pb, hist, sem):
    wid = lax.axis_index("s") * 2 + lax.axis_index("c")

    @pl.loop(0, ROWS_PER_WORKER)
    def _row(r):
        row = wid * ROWS_PER_WORKER + r
        pltpu.async_copy(x_hbm.at[row], keys, sem).wait()

        @pl.loop(0, N, step=L)
        def _flip(j):
            v = keys[pl.ds(j, L)]
            m = lax.shift_right_arithmetic(v, 31) | _MIN_I32
            keys[pl.ds(j, L)] = v ^ m

        for p in range(4):
            shift = 8 * p
            src, dst = (None, pa) if p == 0 else ((pa, pb) if p == 1 else ((pb, pa) if p == 2 else (pa, pb)))

            @pl.loop(0, 256, step=L)
            def _zero(h):
                hist[pl.ds(h, L)] = jnp.zeros((L,), jnp.int32)

            @pl.loop(0, N, step=L)
            def _hist(j):
                k = keys[pl.ds(j, L)]
                dig = lax.shift_right_logical(k, shift) & 255
                cnt, last = plsc.scan_count(dig)
                plsc.addupdate_scatter(hist, [dig], cnt, mask=last)

            @pl.loop(0, 256, step=L, init_carry=np.int32(0))
            def _scan(h, carry):
                v = hist[pl.ds(h, L)]
                c = plsc.cumsum(v)
                hist[pl.ds(h, L)] = c - v + carry
                return carry + jnp.sum(v)

            @pl.loop(0, N, step=L)
            def _perm(j):
                if p == 0:
                    pay = lax.iota(jnp.int32, L) + j
                    k = keys[pl.ds(j, L)]
                else:
                    pay = src[pl.ds(j, L)]
                    k = plsc.load_gather(keys, [pay])
                dig = lax.shift_right_logical(k, shift) & 255
                base = plsc.load_gather(hist, [dig])
                cnt, last = plsc.scan_count(dig)
                plsc.store_scatter(dst, [base + cnt - 1], pay)
                plsc.addupdate_scatter(hist, [dig], cnt, mask=last)

        @pl.loop(0, N, step=L)
        def _vals(j):
            pay = pb[pl.ds(j, L)]
            k = plsc.load_gather(keys, [pay])
            m = (~lax.shift_right_arithmetic(k, 31)) | _MIN_I32
            pa[pl.ds(j, L)] = k ^ m

        pltpu.async_copy(pa, vals_hbm.at[row], sem).wait()
        pltpu.async_copy(pb, idx_hbm.at[row], sem).wait()


@jax.jit
def kernel(x):
    xi = lax.bitcast_convert_type(x, jnp.int32)
    mesh = plsc.VectorSubcoreMesh(core_axis_name="c", subcore_axis_name="s")
    cp = pltpu.CompilerParams()
    if "needs_layout_passes" in pltpu.CompilerParams.__dataclass_fields__:
        cp = dataclasses.replace(cp, needs_layout_passes=False)
    f = pl.kernel(
        _sort_body,
        out_type=(
            jax.ShapeDtypeStruct((ROWS, N), jnp.int32),
            jax.ShapeDtypeStruct((ROWS, N), jnp.int32),
        ),
        mesh=mesh,
        scratch_types=[
            pltpu.VMEM((N,), jnp.int32),
            pltpu.VMEM((N,), jnp.int32),
            pltpu.VMEM((N,), jnp.int32),
            pltpu.VMEM((256,), jnp.int32),
            pltpu.SemaphoreType.DMA,
        ],
        compiler_params=cp,
    )
    vals_i, idx = f(xi)
    return lax.bitcast_convert_type(vals_i, jnp.float32), idx


# speedup vs baseline: 1.8972x; 1.3231x over previous
import dataclasses

import jax
import jax.numpy as jnp
import numpy as np
from jax import lax
from jax.experimental import pallas as pl
from jax.experimental.pallas import tpu as pltpu
from jax.experimental.pallas import tpu_sc as plsc

ROWS = 128
N = 32768
L = 16
NUM_WORKERS = 32
ROWS_PER_WORKER = ROWS // NUM_WORKERS

DIGIT_BITS = (11, 11, 10)
NBINS = 2048

_MIN_I32 = np.int32(-(2**31))


def _digit(k, p):
    shift = sum(DIGIT_BITS[:p])
    mask = (1 << DIGIT_BITS[p]) - 1
    d = k if shift == 0 else lax.shift_right_logical(k, shift)
    return d & mask


def _sort_body(x_hbm, vals_hbm, idx_hbm, keys, pa, pb, hist, sem):
    wid = lax.axis_index("s") * 2 + lax.axis_index("c")

    @pl.loop(0, ROWS_PER_WORKER)
    def _row(r):
        row = wid * ROWS_PER_WORKER + r
        pltpu.async_copy(x_hbm.at[row], keys, sem).wait()

        @pl.loop(0, NBINS, step=L)
        def _zero(h):
            hist[pl.ds(h, L)] = jnp.zeros((L,), jnp.int32)

        @pl.loop(0, N, step=L)
        def _flip(j):
            v = keys[pl.ds(j, L)]
            m = lax.shift_right_arithmetic(v, 31) | _MIN_I32
            f = v ^ m
            keys[pl.ds(j, L)] = f
            dig = _digit(f, 0)
            cnt, last = plsc.scan_count(dig)
            plsc.addupdate_scatter(hist, [dig], cnt, mask=last)

        for p in range(3):
            src, dst = ((None, pa), (pa, pb), (pb, pa))[p]

            if p > 0:
                @pl.loop(0, NBINS, step=L)
                def _zero2(h):
                    hist[pl.ds(h, L)] = jnp.zeros((L,), jnp.int32)

                @pl.loop(0, N, step=L)
                def _hist(j):
                    k = keys[pl.ds(j, L)]
                    dig = _digit(k, p)
                    cnt, last = plsc.scan_count(dig)
                    plsc.addupdate_scatter(hist, [dig], cnt, mask=last)

            @pl.loop(0, NBINS, step=L, init_carry=np.int32(0))
            def _scan(h, carry):
                v = hist[pl.ds(h, L)]
                c = plsc.cumsum(v)
                hist[pl.ds(h, L)] = c - v + carry
                return carry + jnp.sum(v)

            @pl.loop(0, N, step=L)
            def _perm(j):
                if p == 0:
                    pay = lax.iota(jnp.int32, L) + j
                    k = keys[pl.ds(j, L)]
                else:
                    pay = src[pl.ds(j, L)]
                    k = plsc.load_gather(keys, [pay])
                dig = _digit(k, p)
                base = plsc.load_gather(hist, [dig])
                cnt, last = plsc.scan_count(dig)
                plsc.store_scatter(dst, [base + cnt - 1], pay)
                plsc.addupdate_scatter(hist, [dig], cnt, mask=last)

        @pl.loop(0, N, step=L)
        def _vals(j):
            pay = pa[pl.ds(j, L)]
            k = plsc.load_gather(keys, [pay])
            m = (~lax.shift_right_arithmetic(k, 31)) | _MIN_I32
            pb[pl.ds(j, L)] = k ^ m

        pltpu.async_copy(pb, vals_hbm.at[row], sem).wait()
        pltpu.async_copy(pa, idx_hbm.at[row], sem).wait()


@jax.jit
def kernel(x):
    xi = lax.bitcast_convert_type(x, jnp.int32)
    mesh = plsc.VectorSubcoreMesh(core_axis_name="c", subcore_axis_name="s")
    cp = pltpu.CompilerParams()
    if "needs_layout_passes" in pltpu.CompilerParams.__dataclass_fields__:
        cp = dataclasses.replace(cp, needs_layout_passes=False)
    f = pl.kernel(
        _sort_body,
        out_type=(
            jax.ShapeDtypeStruct((ROWS, N), jnp.int32),
            jax.ShapeDtypeStruct((ROWS, N), jnp.int32),
        ),
        mesh=mesh,
        scratch_types=[
            pltpu.VMEM((N,), jnp.int32),
            pltpu.VMEM((N,), jnp.int32),
            pltpu.VMEM((N,), jnp.int32),
            pltpu.VMEM((NBINS,), jnp.int32),
            pltpu.SemaphoreType.DMA,
        ],
        compiler_params=cp,
    )
    vals_i, idx = f(xi)
    return lax.bitcast_convert_type(vals_i, jnp.float32), idx


# speedup vs baseline: 3.2191x; 1.6968x over previous
import dataclasses

import jax
import jax.numpy as jnp
import numpy as np
from jax import lax
from jax.experimental import pallas as pl
from jax.experimental.pallas import tpu as pltpu
from jax.experimental.pallas import tpu_sc as plsc

ROWS = 128
N = 32768
L = 16
NUM_WORKERS = 32
ROWS_PER_WORKER = ROWS // NUM_WORKERS

DIGIT_BITS = (11, 11, 10)
NBINS = 2048

_MIN_I32 = np.int32(-(2**31))


def _digit(k, p):
    shift = sum(DIGIT_BITS[:p])
    mask = (1 << DIGIT_BITS[p]) - 1
    d = k if shift == 0 else lax.shift_right_logical(k, shift)
    return d & mask


def _sort_body(x_hbm, vals_hbm, idx_hbm, keys, pa, pb, hist, sem):
    wid = lax.axis_index("s") * 2 + lax.axis_index("c")

    @pl.loop(0, ROWS_PER_WORKER)
    def _row(r):
        row = wid * ROWS_PER_WORKER + r
        pltpu.async_copy(x_hbm.at[row], keys, sem).wait()

        @pl.loop(0, NBINS, step=L)
        def _zero(h):
            hist[pl.ds(h, L)] = jnp.zeros((L,), jnp.int32)

        def _flip_one(j):
            v = keys[pl.ds(j, L)]
            m = lax.shift_right_arithmetic(v, 31) | _MIN_I32
            f = v ^ m
            keys[pl.ds(j, L)] = f
            return _digit(f, 0)

        dig0 = _flip_one(0)

        @pl.loop(0, N - L, step=L, init_carry=dig0)
        def _flip(j, dig):
            cnt, last = plsc.scan_count(dig)
            dig2 = _flip_one(j + L)
            plsc.addupdate_scatter(hist, [dig], cnt, mask=last)
            return dig2

        cnt0, last0 = plsc.scan_count(_flip)
        plsc.addupdate_scatter(hist, [_flip], cnt0, mask=last0)

        for p in range(3):
            src, dst = ((None, pa), (pa, pb), (pb, pa))[p]

            if p > 0:
                @pl.loop(0, NBINS, step=L)
                def _zero2(h):
                    hist[pl.ds(h, L)] = jnp.zeros((L,), jnp.int32)

                dh0 = _digit(keys[pl.ds(0, L)], p)

                @pl.loop(0, N - L, step=L, init_carry=dh0)
                def _hist(j, dig):
                    cnt, last = plsc.scan_count(dig)
                    dig2 = _digit(keys[pl.ds(j + L, L)], p)
                    plsc.addupdate_scatter(hist, [dig], cnt, mask=last)
                    return dig2

                cnt1, last1 = plsc.scan_count(_hist)
                plsc.addupdate_scatter(hist, [_hist], cnt1, mask=last1)

            @pl.loop(0, NBINS, step=L, init_carry=np.int32(0))
            def _scan(h, carry):
                v = hist[pl.ds(h, L)]
                c = plsc.cumsum(v)
                hist[pl.ds(h, L)] = c - v + carry
                return carry + jnp.sum(v)

            def _pay_dig(j):
                if p == 0:
                    pay = lax.iota(jnp.int32, L) + j
                    k = keys[pl.ds(j, L)]
                else:
                    pay = src[pl.ds(j, L)]
                    k = plsc.load_gather(keys, [pay])
                return pay, _digit(k, p)

            pd0 = _pay_dig(0)

            @pl.loop(0, N - L, step=L, init_carry=pd0)
            def _perm(j, carry):
                pay, dig = carry
                cnt, last = plsc.scan_count(dig)
                nxt = _pay_dig(j + L)
                base = plsc.load_gather(hist, [dig])
                plsc.store_scatter(dst, [base + cnt - 1], pay)
                plsc.addupdate_scatter(hist, [dig], cnt, mask=last)
                return nxt

            payf, digf = _perm
            cntf, lastf = plsc.scan_count(digf)
            basef = plsc.load_gather(hist, [digf])
            plsc.store_scatter(dst, [basef + cntf - 1], payf)
            plsc.addupdate_scatter(hist, [digf], cntf, mask=lastf)

        @pl.loop(0, N, step=L)
        def _vals(j):
            pay = pa[pl.ds(j, L)]
            k = plsc.load_gather(keys, [pay])
            m = (~lax.shift_right_arithmetic(k, 31)) | _MIN_I32
            pb[pl.ds(j, L)] = k ^ m

        pltpu.async_copy(pb, vals_hbm.at[row], sem).wait()
        pltpu.async_copy(pa, idx_hbm.at[row], sem).wait()


@jax.jit
def kernel(x):
    xi = lax.bitcast_convert_type(x, jnp.int32)
    mesh = plsc.VectorSubcoreMesh(core_axis_name="c", subcore_axis_name="s")
    cp = pltpu.CompilerParams()
    if "needs_layout_passes" in pltpu.CompilerParams.__dataclass_fields__:
        cp = dataclasses.replace(cp, needs_layout_passes=False)
    f = pl.kernel(
        _sort_body,
        out_type=(
            jax.ShapeDtypeStruct((ROWS, N), jnp.int32),
            jax.ShapeDtypeStruct((ROWS, N), jnp.int32),
        ),
        mesh=mesh,
        scratch_types=[
            pltpu.VMEM((N,), jnp.int32),
            pltpu.VMEM((N,), jnp.int32),
            pltpu.VMEM((N,), jnp.int32),
            pltpu.VMEM((NBINS,), jnp.int32),
            pltpu.SemaphoreType.DMA,
        ],
        compiler_params=cp,
    )
    vals_i, idx = f(xi)
    return lax.bitcast_convert_type(vals_i, jnp.float32), idx


# speedup vs baseline: 4.1622x; 1.2930x over previous
import dataclasses

import jax
import jax.numpy as jnp
import numpy as np
from jax import lax
from jax.experimental import pallas as pl
from jax.experimental.pallas import tpu as pltpu
from jax.experimental.pallas import tpu_sc as plsc

ROWS = 128
N = 32768
L = 16
NUM_WORKERS = 32
ROWS_PER_WORKER = ROWS // NUM_WORKERS

DIGIT_BITS = (11, 11, 10)
NBINS = 2048

_MIN_I32 = np.int32(-(2**31))


def _digit(k, p):
    shift = sum(DIGIT_BITS[:p])
    mask = (1 << DIGIT_BITS[p]) - 1
    d = k if shift == 0 else lax.shift_right_logical(k, shift)
    return d & mask


def _sort_body(x_hbm, vals_hbm, idx_hbm, keys, pa, pb, h0, h1, h2, sem):
    wid = lax.axis_index("s") * 2 + lax.axis_index("c")

    @pl.loop(0, ROWS_PER_WORKER)
    def _row(r):
        row = wid * ROWS_PER_WORKER + r
        pltpu.async_copy(x_hbm.at[row], keys, sem).wait()

        hists = (h0, h1, h2)

        @pl.loop(0, NBINS, step=L)
        def _zero(h):
            z = jnp.zeros((L,), jnp.int32)
            h0[pl.ds(h, L)] = z
            h1[pl.ds(h, L)] = z

        @pl.loop(0, NBINS // 2, step=L)
        def _zero2(h):
            h2[pl.ds(h, L)] = jnp.zeros((L,), jnp.int32)

        def _flip_one(j):
            v = keys[pl.ds(j, L)]
            m = lax.shift_right_arithmetic(v, 31) | _MIN_I32
            f = v ^ m
            keys[pl.ds(j, L)] = f
            return _digit(f, 0), _digit(f, 1), _digit(f, 2)

        def _hist_upd(digs):
            cls = [plsc.scan_count(d) for d in digs]
            for hr, d, (cnt, last) in zip(hists, digs, cls):
                plsc.addupdate_scatter(hr, [d], cnt, mask=last)

        digs0 = _flip_one(0)

        @pl.loop(0, N - L, step=L, init_carry=digs0)
        def _flip(j, digs):
            c0, l0 = plsc.scan_count(digs[0])
            c1, l1 = plsc.scan_count(digs[1])
            c2, l2 = plsc.scan_count(digs[2])
            nxt = _flip_one(j + L)
            plsc.addupdate_scatter(h0, [digs[0]], c0, mask=l0)
            plsc.addupdate_scatter(h1, [digs[1]], c1, mask=l1)
            plsc.addupdate_scatter(h2, [digs[2]], c2, mask=l2)
            return nxt

        _hist_upd(_flip)

        for p in range(3):
            src, dst = ((None, pa), (pa, pb), (pb, pa))[p]
            hist = hists[p]
            nb = NBINS if p < 2 else NBINS // 2

            @pl.loop(0, nb, step=L, init_carry=np.int32(0))
            def _scan(h, carry):
                v = hist[pl.ds(h, L)]
                c = plsc.cumsum(v)
                hist[pl.ds(h, L)] = c - v + carry
                return carry + jnp.sum(v)

            def _pay_dig(j):
                if p == 0:
                    pay = lax.iota(jnp.int32, L) + j
                    k = keys[pl.ds(j, L)]
                else:
                    pay = src[pl.ds(j, L)]
                    k = plsc.load_gather(keys, [pay])
                return pay, _digit(k, p)

            pd0 = _pay_dig(0)

            @pl.loop(0, N - L, step=L, init_carry=pd0)
            def _perm(j, carry):
                pay, dig = carry
                cnt, last = plsc.scan_count(dig)
                nxt = _pay_dig(j + L)
                base = plsc.load_gather(hist, [dig])
                plsc.store_scatter(dst, [base + cnt - 1], pay)
                plsc.addupdate_scatter(hist, [dig], cnt, mask=last)
                return nxt

            payf, digf = _perm
            cntf, lastf = plsc.scan_count(digf)
            basef = plsc.load_gather(hist, [digf])
            plsc.store_scatter(dst, [basef + cntf - 1], payf)
            plsc.addupdate_scatter(hist, [digf], cntf, mask=lastf)

        @pl.loop(0, N, step=L)
        def _vals(j):
            pay = pa[pl.ds(j, L)]
            k = plsc.load_gather(keys, [pay])
            m = (~lax.shift_right_arithmetic(k, 31)) | _MIN_I32
            pb[pl.ds(j, L)] = k ^ m

        pltpu.async_copy(pb, vals_hbm.at[row], sem).wait()
        pltpu.async_copy(pa, idx_hbm.at[row], sem).wait()


@jax.jit
def kernel(x):
    xi = lax.bitcast_convert_type(x, jnp.int32)
    mesh = plsc.VectorSubcoreMesh(core_axis_name="c", subcore_axis_name="s")
    cp = pltpu.CompilerParams()
    if "needs_layout_passes" in pltpu.CompilerParams.__dataclass_fields__:
        cp = dataclasses.replace(cp, needs_layout_passes=False)
    f = pl.kernel(
        _sort_body,
        out_type=(
            jax.ShapeDtypeStruct((ROWS, N), jnp.int32),
            jax.ShapeDtypeStruct((ROWS, N), jnp.int32),
        ),
        mesh=mesh,
        scratch_types=[
            pltpu.VMEM((N,), jnp.int32),
            pltpu.VMEM((N,), jnp.int32),
            pltpu.VMEM((N,), jnp.int32),
            pltpu.VMEM((NBINS,), jnp.int32),
            pltpu.VMEM((NBINS,), jnp.int32),
            pltpu.VMEM((NBINS // 2,), jnp.int32),
            pltpu.SemaphoreType.DMA,
        ],
        compiler_params=cp,
    )
    vals_i, idx = f(xi)
    return lax.bitcast_convert_type(vals_i, jnp.float32), idx
